# baseline (speedup 1.0000x reference)
import functools

import jax
import jax.numpy as jnp
from jax import lax
from jax.experimental import pallas as pl
from jax.experimental.pallas import tpu as pltpu
from jax.experimental.pallas import tpu_sc as plsc

_N = 10000
_D = 128
_E = 320000
_C = 128
_CPT = 160
_EPT = _CPT * _C
_EPAD = 16 * _EPT
_PW = 3 * _C
_AR = 10240
_ZR = _AR // 16
_R = 1000
_G = _N // _R


def _lrelu(v):
    return jnp.where(v >= 0, v, 0.01 * v)


def _dot(a, b):
    return jnp.dot(a, b, preferred_element_type=jnp.float32)


def _mlp_in(np8, wn, bn, wi, bi):
    def body(np_r, wn_r, bn_r, wi_r, bi_r, o_r):
        h = _lrelu(_dot(np_r[...], wn_r[...]) + bn_r[...])
        o_r[...] = _lrelu(_dot(h, wi_r[...]) + bi_r[...])

    return pl.pallas_call(
        body,
        grid=(_G,),
        in_specs=[
            pl.BlockSpec((_R, 8), lambda i: (i, 0)),
            pl.BlockSpec((8, _D), lambda i: (0, 0)),
            pl.BlockSpec((1, _D), lambda i: (0, 0)),
            pl.BlockSpec((_D, _D), lambda i: (0, 0)),
            pl.BlockSpec((1, _D), lambda i: (0, 0)),
        ],
        out_specs=pl.BlockSpec((_R, _D), lambda i: (i, 0)),
        out_shape=jax.ShapeDtypeStruct((_N, _D), jnp.float32),
    )(np8, wn, bn, wi, bi)


def _combine(x, s0, s1, cnt0, cnt1, wroot, wrel, b, tail_args=None):
    tail = tail_args is not None

    def body(x_r, s0_r, s1_r, c0_r, c1_r, wroot_r, wrel_r, b_r, *rest):
        if tail:
            wo1_r, bo1_r, wo2_r, bo2_r, o_r = rest
        else:
            (o_r,) = rest
        wrel_v = wrel_r[...]
        inv0 = 1.0 / jnp.maximum(c0_r[...][:, 0:1], 1.0)
        inv1 = 1.0 / jnp.maximum(c1_r[...][:, 0:1], 1.0)
        y = _dot(x_r[...], wroot_r[...]) + b_r[...]
        y = y + _dot(s0_r[...] * inv0, wrel_v[0])
        y = y + _dot(s1_r[...] * inv1, wrel_v[1])
        if tail:
            z = _lrelu(_dot(y, wo1_r[...]) + bo1_r[...])
            y = _dot(z, wo2_r[...]) + bo2_r[...]
        o_r[...] = y

    def full(shape):
        return pl.BlockSpec(shape, lambda i: tuple(0 for _ in shape))

    in_specs = [
        pl.BlockSpec((_R, _D), lambda i: (i, 0)),
        pl.BlockSpec((_R, _D), lambda i: (i, 0)),
        pl.BlockSpec((_R, _D), lambda i: (i, 0)),
        pl.BlockSpec((_R, 16), lambda i: (i, 0)),
        pl.BlockSpec((_R, 16), lambda i: (i, 0)),
        full((_D, _D)),
        full((2, _D, _D)),
        full((1, _D)),
    ]
    args = [x, s0, s1, cnt0, cnt1, wroot, wrel, b]
    if tail:
        in_specs += [full((_D, _D)), full((1, _D)),
                     full((_D, _D)), full((1, _D))]
        args += list(tail_args)
    return pl.pallas_call(
        body,
        grid=(_G,),
        in_specs=in_specs,
        out_specs=pl.BlockSpec((_R, _D), lambda i: (i, 0)),
        out_shape=jax.ShapeDtypeStruct((_N, _D), jnp.float32),
    )(*args)


def _sc_segsum(x, pack, zrows, zrow1, ones1, with_counts):
    mesh = plsc.VectorSubcoreMesh(core_axis_name="c", subcore_axis_name="s")
    f32, i32 = jnp.float32, jnp.int32
    NB = 2

    out_type = [
        jax.ShapeDtypeStruct((_AR, _D), f32),
        jax.ShapeDtypeStruct((_AR, _D), f32),
    ]
    scratch = (
        [pltpu.VMEM((_PW,), i32)] * NB
        + [pltpu.VMEM((_C,), i32)] * NB
        + [pltpu.VMEM((_C,), i32)] * NB
        + [pltpu.VMEM((_C, _D), f32)] * NB
        + [pltpu.VMEM_SHARED((_AR, _D), f32)]
        + [pltpu.SemaphoreType.DMA] * (3 * NB)
    )
    if with_counts:
        out_type += [jax.ShapeDtypeStruct((_AR,), f32),
                     jax.ShapeDtypeStruct((_AR,), f32)]
        scratch += [pltpu.VMEM((_C,), f32),
                    pltpu.VMEM_SHARED((_AR,), f32)]

    @functools.partial(pl.kernel, mesh=mesh, out_type=out_type,
                       scratch_types=scratch)
    def k(*refs):
        x_h, pack_h, z_h = refs[0], refs[1], refs[2]
        if with_counts:
            z1_h, ones_h = refs[3], refs[4]
            s0_h, s1_h, cnt0_h, cnt1_h = refs[5:9]
            rest = refs[9:]
        else:
            s0_h, s1_h = refs[3:5]
            rest = refs[5:]
        pk = rest[0:NB]
        gix = rest[NB:2 * NB]
        eff = rest[2 * NB:3 * NB]
        rows = rest[3 * NB:4 * NB]
        acc = rest[4 * NB]
        isem = rest[4 * NB + 1:4 * NB + 1 + NB]
        gsem = rest[4 * NB + 1 + NB:4 * NB + 1 + 2 * NB]
        ssem = rest[4 * NB + 1 + 2 * NB:4 * NB + 1 + 3 * NB]
        if with_counts:
            ones_v, cacc = rest[4 * NB + 1 + 3 * NB:]
        c = lax.axis_index("c")
        s = lax.axis_index("s")
        pltpu.sync_copy(z_h, acc.at[pl.ds(s * _ZR, _ZR)])
        if with_counts:
            pltpu.sync_copy(z1_h, cacc.at[pl.ds(s * _ZR, _ZR)])
            pltpu.sync_copy(ones_h, ones_v)
        plsc.subcore_barrier()

        def cbase(i):
            return pl.multiple_of((s * _CPT + i) * _PW, _C)

        for b in range(NB):
            pltpu.async_copy(pack_h.at[pl.ds(cbase(b), _PW)], pk[b],
                             isem[b])

        def body(g, carry):
            i0 = NB * g
            for b in range(NB):
                pltpu.make_async_copy(pack_h.at[pl.ds(0, _PW)], pk[b],
                                      isem[b]).wait()

                @pl.when(g > 0)
                def _(b=b):
                    pltpu.make_async_copy(rows[b], acc.at[eff[b]],
                                          ssem[b]).wait()
                    if with_counts:
                        pltpu.make_async_copy(ones_v, cacc.at[eff[b]],
                                              ssem[b]).wait()

                for j in range(_C // 16):
                    sl = pl.ds(j * 16, 16)
                    gix[b][sl] = pk[b][pl.ds(j * 16, 16)]
                    eff[b][sl] = jnp.where(
                        pk[b][pl.ds(2 * _C + j * 16, 16)] == c,
                        pk[b][pl.ds(_C + j * 16, 16)], _N)
                pltpu.async_copy(x_h.at[gix[b]], rows[b], gsem[b])
            for b in range(NB):
                pltpu.make_async_copy(x_h.at[gix[b]], rows[b],
                                      gsem[b]).wait()
                pltpu.async_copy(rows[b], acc.at[eff[b]], ssem[b],
                                add=True)
                if with_counts:
                    pltpu.async_copy(ones_v, cacc.at[eff[b]], ssem[b],
                                    add=True)
                pltpu.async_copy(pack_h.at[pl.ds(cbase(i0 + NB + b), _PW)],
                                 pk[b], isem[b])
            return carry

        lax.fori_loop(0, _CPT // NB, body, 0)
        for b in range(NB):
            pltpu.make_async_copy(pack_h.at[pl.ds(0, _PW)], pk[b],
                                  isem[b]).wait()
            pltpu.make_async_copy(rows[b], acc.at[eff[b]], ssem[b]).wait()
            if with_counts:
                pltpu.make_async_copy(ones_v, cacc.at[eff[b]],
                                      ssem[b]).wait()
        plsc.subcore_barrier()

        @pl.when(c == 0)
        def _():
            pltpu.sync_copy(acc.at[pl.ds(s * _ZR, _ZR)],
                            s0_h.at[pl.ds(s * _ZR, _ZR)])
            if with_counts:
                pltpu.sync_copy(cacc.at[pl.ds(s * _ZR, _ZR)],
                                cnt0_h.at[pl.ds(s * _ZR, _ZR)])

        @pl.when(c == 1)
        def _():
            pltpu.sync_copy(acc.at[pl.ds(s * _ZR, _ZR)],
                            s1_h.at[pl.ds(s * _ZR, _ZR)])
            if with_counts:
                pltpu.sync_copy(cacc.at[pl.ds(s * _ZR, _ZR)],
                                cnt1_h.at[pl.ds(s * _ZR, _ZR)])

    if with_counts:
        return k(x, pack, zrows, zrow1, ones1)
    return k(x, pack, zrows)


def kernel(des, tweet, num_prop, cat_prop, W_num, b_num, W_in, b_in, W_rel,
           W_root, b_rgcn, W_out1, b_out1, W_out2, b_out2, edge_index,
           edge_type):
    f32 = jnp.float32
    np8 = jnp.pad(num_prop, ((0, 0), (0, 2)))
    wn8 = jnp.pad(W_num, ((0, 2), (0, 0)))
    bn = b_num.reshape(1, _D)
    bi = b_in.reshape(1, _D)
    br = b_rgcn.reshape(1, _D)
    bo1 = b_out1.reshape(1, _D)
    wo2 = jnp.pad(W_out2, ((0, 0), (0, _D - 2)))
    bo2 = jnp.pad(b_out2, (0, _D - 2)).reshape(1, _D)

    pad = _EPAD - _E
    srcp = jnp.concatenate([edge_index[0], jnp.zeros((pad,), jnp.int32)])
    dstp = jnp.concatenate([edge_index[1], jnp.zeros((pad,), jnp.int32)])
    typp = jnp.concatenate([edge_type, jnp.full((pad,), 2, jnp.int32)])
    pack = jnp.stack([srcp.reshape(-1, _C), dstp.reshape(-1, _C),
                      typp.reshape(-1, _C)], axis=1).reshape(-1)
    pack = jnp.concatenate([pack, jnp.zeros((4 * _PW,), jnp.int32)])
    zrows = jnp.zeros((_ZR, _D), f32)
    zrow1 = jnp.zeros((_ZR,), f32)
    ones1 = jnp.ones((_C,), f32)

    x0 = _mlp_in(np8, wn8, bn, W_in, bi)
    s0a, s1a, c0f, c1f = _sc_segsum(x0, pack, zrows, zrow1, ones1,
                                    with_counts=True)
    cnt0 = jnp.broadcast_to(c0f[:, None], (_AR, 16))
    cnt1 = jnp.broadcast_to(c1f[:, None], (_AR, 16))
    x1 = _combine(x0, s0a, s1a, cnt0, cnt1, W_root, W_rel, br)
    s0b, s1b = _sc_segsum(x1, pack, zrows, zrow1, ones1,
                          with_counts=False)
    out = _combine(x1, s0b, s1b, cnt0, cnt1, W_root, W_rel, br,
                   tail_args=(W_out1, bo1, wo2, bo2))
    return out[:, :2]
